# baseline (speedup 1.0000x reference)
import functools

import jax
import jax.numpy as jnp
from jax import lax
from jax.experimental import pallas as pl
from jax.experimental.pallas import tpu as pltpu
from jax.experimental.pallas import tpu_sc as plsc

NUM_CORES = 2
NUM_SUBCORES = 16
NUM_WORKERS = NUM_CORES * NUM_SUBCORES

VOCAB = 1000000
B_BATCH = 4096
SEQ = 200
DIM = 64
PDIM = 128

TBLK = 128
N_FULL_BLOCKS = VOCAB // TBLK
TAIL = VOCAB - N_FULL_BLOCKS * TBLK
MAX_BLOCKS_PER_W = (N_FULL_BLOCKS + NUM_WORKERS - 1) // NUM_WORKERS

B_PER_W = B_BATCH // NUM_WORKERS
ROWS_PER_CHUNK = 2
CHUNK = ROWS_PER_CHUNK * SEQ
N_CHUNKS = B_PER_W // ROWS_PER_CHUNK
GATHER_SPLITS = tuple((o, 64) for o in range(0, 384, 64)) + ((384, 16),)

_TILED = pltpu.CompilerParams(use_tc_tiling_on_sc=True)
_TILED_NOLAYOUT = pltpu.CompilerParams(
    use_tc_tiling_on_sc=True, needs_layout_passes=False
)


def _mesh():
    return plsc.VectorSubcoreMesh(core_axis_name="c", subcore_axis_name="s")


@jax.jit
def _embedding_gather(token_ids_flat, table):
    @functools.partial(
        pl.kernel,
        mesh=_mesh(),
        out_type=jax.ShapeDtypeStruct((B_BATCH, SEQ, PDIM), jnp.float32),
        scratch_types=[
            pltpu.VMEM((CHUNK,), jnp.int32),
            pltpu.VMEM((CHUNK,), jnp.int32),
            pltpu.VMEM((CHUNK, PDIM), jnp.float32),
            pltpu.VMEM((CHUNK, PDIM), jnp.float32),
            pltpu.SemaphoreType.DMA,
            pltpu.SemaphoreType.DMA,
            pltpu.SemaphoreType.DMA,
        ],
        compiler_params=_TILED,
    )
    def k(idx_hbm, table_hbm, out_hbm, idx0, idx1, rows0, rows1,
          sem_g, sem_w0, sem_w1):
        wid = lax.axis_index("s") * NUM_CORES + lax.axis_index("c")
        tok_base = wid * (B_PER_W * SEQ)
        b_base = wid * B_PER_W

        def gathers(idx_v, rows_v):
            return [
                pltpu.make_async_copy(
                    table_hbm.at[idx_v.at[pl.ds(g0, glen)]],
                    rows_v.at[pl.ds(g0, glen)],
                    sem_g,
                )
                for g0, glen in GATHER_SPLITS
            ]

        def writes(rows_v, sem_w, j):
            b0 = b_base + j * ROWS_PER_CHUNK
            return [
                pltpu.make_async_copy(
                    rows_v.at[pl.ds(r * SEQ, SEQ)],
                    out_hbm.at[b0 + r],
                    sem_w,
                )
                for r in range(ROWS_PER_CHUNK)
            ]

        def load_idx(idx_v, j):
            pltpu.sync_copy(
                idx_hbm.at[pl.ds(tok_base + j * CHUNK, CHUNK)], idx_v
            )

        bufs = ((idx0, rows0, sem_w0), (idx1, rows1, sem_w1))

        def do_chunk(j, slot, first2, last):
            idx_v, rows_v, sem_w = bufs[slot]
            idx_n = bufs[1 - slot][0]
            if not first2:
                for c in writes(rows_v, sem_w, j - 2):
                    c.wait()
            for c in gathers(idx_v, rows_v):
                c.start()
            if not last:
                load_idx(idx_n, j + 1)
            for c in gathers(idx_v, rows_v):
                c.wait()
            for c in writes(rows_v, sem_w, j):
                c.start()

        def body(i, carry):
            do_chunk(2 * i, 0, False, False)
            do_chunk(2 * i + 1, 1, False, False)
            return carry

        load_idx(idx0, 0)
        do_chunk(0, 0, True, False)
        do_chunk(1, 1, True, False)
        lax.fori_loop(1, N_CHUNKS // 2 - 1, body, 0)
        do_chunk(N_CHUNKS - 2, 0, False, False)
        do_chunk(N_CHUNKS - 1, 1, False, True)
        for c in writes(rows0, sem_w0, N_CHUNKS - 2):
            c.wait()
        for c in writes(rows1, sem_w1, N_CHUNKS - 1):
            c.wait()

    return k(token_ids_flat, table)


def kernel(token_ids, weight):
    s0, s1 = token_ids.shape
    flat = token_ids.reshape(s0 * s1)
    table = jnp.pad(weight, ((0, 0), (0, PDIM - DIM)))
    out = _embedding_gather(flat, table)
    return out[:, :, :DIM]


# speedup vs baseline: 1.0007x; 1.0007x over previous
import functools

import jax
import jax.numpy as jnp
from jax import lax
from jax.experimental import pallas as pl
from jax.experimental.pallas import tpu as pltpu
from jax.experimental.pallas import tpu_sc as plsc

NUM_CORES = 2
NUM_SUBCORES = 16
NUM_WORKERS = NUM_CORES * NUM_SUBCORES

B_BATCH = 4096
SEQ = 200
DIM = 64
PDIM = 128
B_PER_W = B_BATCH // NUM_WORKERS
ROWS_PER_CHUNK = 2
CHUNK = ROWS_PER_CHUNK * SEQ
N_CHUNKS = B_PER_W // ROWS_PER_CHUNK
GATHER_SPLITS = ((0, 128), (128, 128), (256, 128), (384, 16))


@jax.jit
def _embedding_gather(token_ids_flat, table):
    mesh = plsc.VectorSubcoreMesh(core_axis_name="c", subcore_axis_name="s")

    @functools.partial(
        pl.kernel,
        mesh=mesh,
        out_type=jax.ShapeDtypeStruct((B_BATCH, SEQ, PDIM), jnp.float32),
        scratch_types=[
            pltpu.VMEM((CHUNK,), jnp.int32),
            pltpu.VMEM((CHUNK,), jnp.int32),
            pltpu.VMEM((CHUNK, PDIM), jnp.float32),
            pltpu.VMEM((CHUNK, PDIM), jnp.float32),
            pltpu.SemaphoreType.DMA,
            pltpu.SemaphoreType.DMA,
            pltpu.SemaphoreType.DMA,
        ],
        compiler_params=pltpu.CompilerParams(use_tc_tiling_on_sc=True),
    )
    def k(idx_hbm, table_hbm, out_hbm, idx0, idx1, rows0, rows1,
          sem_g, sem_w0, sem_w1):
        wid = lax.axis_index("s") * NUM_CORES + lax.axis_index("c")
        tok_base = wid * (B_PER_W * SEQ)
        b_base = wid * B_PER_W

        def gathers(idx_v, rows_v):
            return [
                pltpu.make_async_copy(
                    table_hbm.at[idx_v.at[pl.ds(g0, glen)]],
                    rows_v.at[pl.ds(g0, glen)],
                    sem_g,
                )
                for g0, glen in GATHER_SPLITS
            ]

        def writes(rows_v, sem_w, j):
            b0 = b_base + j * ROWS_PER_CHUNK
            return [
                pltpu.make_async_copy(
                    rows_v.at[pl.ds(r * SEQ, SEQ)],
                    out_hbm.at[b0 + r],
                    sem_w,
                )
                for r in range(ROWS_PER_CHUNK)
            ]

        def load_idx(idx_v, j):
            pltpu.sync_copy(
                idx_hbm.at[pl.ds(tok_base + j * CHUNK, CHUNK)], idx_v
            )

        bufs = ((idx0, rows0, sem_w0), (idx1, rows1, sem_w1))

        def do_chunk(j, slot, first2, last):
            idx_v, rows_v, sem_w = bufs[slot]
            idx_n = bufs[1 - slot][0]
            if not first2:
                for c in writes(rows_v, sem_w, j - 2):
                    c.wait()
            for c in gathers(idx_v, rows_v):
                c.start()
            if not last:
                load_idx(idx_n, j + 1)
            for c in gathers(idx_v, rows_v):
                c.wait()
            for c in writes(rows_v, sem_w, j):
                c.start()

        def body(i, carry):
            do_chunk(2 * i, 0, False, False)
            do_chunk(2 * i + 1, 1, False, False)
            return carry

        load_idx(idx0, 0)
        do_chunk(0, 0, True, False)
        do_chunk(1, 1, True, False)
        lax.fori_loop(1, N_CHUNKS // 2 - 1, body, 0)
        do_chunk(N_CHUNKS - 2, 0, False, False)
        do_chunk(N_CHUNKS - 1, 1, False, True)
        for c in writes(rows0, sem_w0, N_CHUNKS - 2):
            c.wait()
        for c in writes(rows1, sem_w1, N_CHUNKS - 1):
            c.wait()

    return k(token_ids_flat, table)


def kernel(token_ids, weight):
    s0, s1 = token_ids.shape
    flat = token_ids.reshape(s0 * s1)
    table = jnp.pad(weight, ((0, 0), (0, PDIM - DIM)))
    out = _embedding_gather(flat, table)
    return out[:, :, :DIM]
